# baseline (speedup 1.0000x reference)
import functools

import jax
import jax.numpy as jnp
from jax import lax
from jax.experimental import pallas as pl
from jax.experimental.pallas import tpu as pltpu
from jax.experimental.pallas import tpu_sc as plsc

NUM_ROWS = 100000
LATENT_DIM = 64
XIJ_DIM = 16
BATCH = 16384
WIDTH = 128

_info = plsc.get_sparse_core_info()
NC, NS, L = _info.num_cores, _info.num_subcores, _info.num_lanes
NW = NC * NS
BPW = BATCH // NW
CHUNK = 128
NCH = BPW // CHUNK


def _sc_body(users_hbm, items_hbm, xij_hbm, ut_hbm, it_hbm, out_hbm,
             uidx_v, iidx_v, xij_v, ubuf, ibuf, out_v, sem0, sem1):
    wid = lax.axis_index("s") * NC + lax.axis_index("c")
    base = wid * BPW

    for k in range(NCH):
        pltpu.sync_copy(users_hbm.at[pl.ds(base + k * CHUNK, CHUNK)], uidx_v.at[k])
        pltpu.sync_copy(items_hbm.at[pl.ds(base + k * CHUNK, CHUNK)], iidx_v.at[k])
    pltpu.sync_copy(xij_hbm.at[pl.ds(base, BPW)], xij_v)

    sems = (sem0, sem1)

    def fire(k):
        par = k % 2
        s = sems[par]
        return (pltpu.async_copy(ut_hbm.at[uidx_v.at[k]], ubuf.at[par], s),
                pltpu.async_copy(it_hbm.at[iidx_v.at[k]], ibuf.at[par], s))

    inflight = fire(0)

    for k in range(NCH):
        par = k % 2
        nxt = fire(k + 1) if k + 1 < NCH else None
        for c in inflight:
            c.wait()
        inflight = nxt

        ub = ubuf.at[par]
        ib = ibuf.at[par]
        cb = k * CHUNK
        lane = lax.iota(jnp.int32, L)

        @plsc.parallel_loop(0, CHUNK // L, 1, unroll=4)
        def group_body(g):
            gb = g * L
            xg = xij_v[pl.ds(cb + gb, L)]
            d_acc = jnp.zeros((L,), jnp.float32)
            s_acc = jnp.ones((L,), jnp.float32)
            for i in range(L):
                r = gb + i
                x = xg[i]
                us = [ub[r, pl.ds(j * L, L)] for j in range(5)]
                ivs = [ib[r, pl.ds(j * L, L)] for j in range(4)]
                v1 = ib[r, pl.ds(64, L)]
                v0 = ib[r, pl.ds(80, L)]
                ivs.append(v0 + x * (v1 - v0))
                es = [jnp.exp(u) for u in us]
                en = [jnp.exp(-iv) for iv in ivs]
                rec = [1.0 / (1.0 + a) for a in en]
                ds = [es[j] * rec[j] for j in range(5)]
                s_v = (es[0] + es[1]) + (es[2] + es[3]) + es[4]
                d_v = (ds[0] + ds[1]) + (ds[2] + ds[3]) + ds[4]
                d_acc = jnp.where(lane == i, jnp.sum(d_v), d_acc)
                s_acc = jnp.where(lane == i, jnp.sum(s_v), s_acc)
            out_v[pl.ds(cb + gb, L)] = d_acc / s_acc

    pltpu.sync_copy(out_v, out_hbm.at[pl.ds(base, BPW)])


@jax.jit
def _run(users, items, xij, W_user, W_item, W_user_xij, W_item_xij1, W_item_xij0):
    hi = jax.lax.Precision.HIGH
    xu = jnp.concatenate([W_user, W_user_xij], axis=1)
    xi = jnp.concatenate([W_item, W_item_xij1, W_item_xij0], axis=1)
    e_u = jnp.eye(LATENT_DIM + XIJ_DIM, WIDTH, dtype=jnp.float32)
    e_i = jnp.eye(LATENT_DIM + 2 * XIJ_DIM, WIDTH, dtype=jnp.float32)
    ut = jnp.dot(xu, e_u, precision=hi)
    it = jnp.dot(xi, e_i, precision=hi)

    mesh = plsc.VectorSubcoreMesh(core_axis_name="c", subcore_axis_name="s")
    f = pl.kernel(
        _sc_body,
        mesh=mesh,
        compiler_params=pltpu.CompilerParams(needs_layout_passes=False),
        out_type=jax.ShapeDtypeStruct((BATCH,), jnp.float32),
        scratch_types=[
            pltpu.VMEM((NCH, CHUNK), jnp.int32),
            pltpu.VMEM((NCH, CHUNK), jnp.int32),
            pltpu.VMEM((BPW,), jnp.float32),
            pltpu.VMEM((2, CHUNK, WIDTH), jnp.float32),
            pltpu.VMEM((2, CHUNK, WIDTH), jnp.float32),
            pltpu.VMEM((BPW,), jnp.float32),
            pltpu.SemaphoreType.DMA,
            pltpu.SemaphoreType.DMA,
        ],
    )
    return f(users, items, xij, ut, it)


def kernel(users, items, xij, W_user, W_item, W_user_xij, W_item_xij1, W_item_xij0):
    return _run(users, items, xij, W_user, W_item, W_user_xij,
                W_item_xij1, W_item_xij0)


# speedup vs baseline: 1.1881x; 1.1881x over previous
import functools

import jax
import jax.numpy as jnp
from jax import lax
from jax.experimental import pallas as pl
from jax.experimental.pallas import tpu as pltpu
from jax.experimental.pallas import tpu_sc as plsc

NUM_ROWS = 100000
LATENT_DIM = 64
XIJ_DIM = 16
BATCH = 16384
WIDTH = 128

_info = plsc.get_sparse_core_info()
NC, NS, L = _info.num_cores, _info.num_subcores, _info.num_lanes
NW = NC * NS
BPW = BATCH // NW
CHUNK = 128
NCH = BPW // CHUNK


def _sc_body(users_hbm, items_hbm, xij_hbm, ut_hbm, it_hbm, out_hbm,
             uidx_v, iidx_v, xij_v, ubuf, ibuf, out_v, sem0, sem1):
    wid = lax.axis_index("s") * NC + lax.axis_index("c")
    base = wid * BPW

    for k in range(NCH):
        pltpu.sync_copy(users_hbm.at[pl.ds(base + k * CHUNK, CHUNK)], uidx_v.at[k])
        pltpu.sync_copy(items_hbm.at[pl.ds(base + k * CHUNK, CHUNK)], iidx_v.at[k])
    pltpu.sync_copy(xij_hbm.at[pl.ds(base, BPW)], xij_v)

    sems = (sem0, sem1)

    def fire(k):
        par = k % 2
        s = sems[par]
        return (pltpu.async_copy(ut_hbm.at[uidx_v.at[k]], ubuf.at[par], s),
                pltpu.async_copy(it_hbm.at[iidx_v.at[k]], ibuf.at[par], s))

    inflight = fire(0)

    for k in range(NCH):
        par = k % 2
        nxt = fire(k + 1) if k + 1 < NCH else None
        for c in inflight:
            c.wait()
        inflight = nxt

        ub = ubuf.at[par]
        ib = ibuf.at[par]
        cb = k * CHUNK
        lane = lax.iota(jnp.int32, L)

        @plsc.parallel_loop(0, CHUNK // L, 1, unroll=2)
        def group_body(g):
            gb = g * L
            xg = xij_v[pl.ds(cb + gb, L)]
            d_acc = jnp.zeros((L,), jnp.float32)
            s_acc = jnp.ones((L,), jnp.float32)
            for i in range(L):
                r = gb + i
                x = xg[i]
                us = [ub[r, pl.ds(j * L, L)] for j in range(5)]
                ivs = [ib[r, pl.ds(j * L, L)] for j in range(4)]
                v1 = ib[r, pl.ds(64, L)]
                v0 = ib[r, pl.ds(80, L)]
                ivs.append(v0 + x * (v1 - v0))
                es = [jnp.exp(u) for u in us]
                en = [jnp.exp(-iv) for iv in ivs]
                rec = [1.0 / (1.0 + a) for a in en]
                ds = [es[j] * rec[j] for j in range(5)]
                s_v = (es[0] + es[1]) + (es[2] + es[3]) + es[4]
                d_v = (ds[0] + ds[1]) + (ds[2] + ds[3]) + ds[4]
                d_acc = jnp.where(lane == i, jnp.sum(d_v), d_acc)
                s_acc = jnp.where(lane == i, jnp.sum(s_v), s_acc)
            out_v[pl.ds(cb + gb, L)] = d_acc / s_acc

    pltpu.sync_copy(out_v, out_hbm.at[pl.ds(base, BPW)])


@jax.jit
def _run(users, items, xij, W_user, W_item, W_user_xij, W_item_xij1, W_item_xij0):
    hi = jax.lax.Precision.HIGH
    xu = jnp.concatenate([W_user, W_user_xij], axis=1)
    xi = jnp.concatenate([W_item, W_item_xij1, W_item_xij0], axis=1)
    e_u = jnp.eye(LATENT_DIM + XIJ_DIM, WIDTH, dtype=jnp.float32)
    e_i = jnp.eye(LATENT_DIM + 2 * XIJ_DIM, WIDTH, dtype=jnp.float32)
    ut = jnp.dot(xu, e_u, precision=hi)
    it = jnp.dot(xi, e_i, precision=hi)

    mesh = plsc.VectorSubcoreMesh(core_axis_name="c", subcore_axis_name="s")
    f = pl.kernel(
        _sc_body,
        mesh=mesh,
        compiler_params=pltpu.CompilerParams(needs_layout_passes=False),
        out_type=jax.ShapeDtypeStruct((BATCH,), jnp.float32),
        scratch_types=[
            pltpu.VMEM((NCH, CHUNK), jnp.int32),
            pltpu.VMEM((NCH, CHUNK), jnp.int32),
            pltpu.VMEM((BPW,), jnp.float32),
            pltpu.VMEM((2, CHUNK, WIDTH), jnp.float32),
            pltpu.VMEM((2, CHUNK, WIDTH), jnp.float32),
            pltpu.VMEM((BPW,), jnp.float32),
            pltpu.SemaphoreType.DMA,
            pltpu.SemaphoreType.DMA,
        ],
    )
    return f(users, items, xij, ut, it)


def kernel(users, items, xij, W_user, W_item, W_user_xij, W_item_xij1, W_item_xij0):
    return _run(users, items, xij, W_user, W_item, W_user_xij,
                W_item_xij1, W_item_xij0)
